# baseline (speedup 1.0000x reference)
import functools

import jax
import jax.numpy as jnp
from jax import lax
from jax.experimental import pallas as pl
from jax.experimental.pallas import tpu as pltpu
from jax.experimental.pallas import tpu_sc as plsc

N = 10000
E = 160000
D = 256
K = 10
ALPHA = 0.1
BETA = 0.9
_s = sum(BETA**j for j in range(K))
THETA = BETA**K + ALPHA * _s

NP = 10240
R = 80
NBLK = NP // R
NTILES = 16
EPW = E // NTILES
BLOCKW = (R + 1) * NP
PER_TILE_W = BLOCKW // NTILES


def _sc_body(row_hbm, col_hbm, c_out, deg_out, rowv, colv, idxv, onesv,
             zerosv, blk_sh, deg_sh):
    c = lax.axis_index("c")
    s = lax.axis_index("s")

    pltpu.sync_copy(row_hbm.at[pl.ds(s * EPW, EPW)], rowv)
    pltpu.sync_copy(col_hbm.at[pl.ds(s * EPW, EPW)], colv)

    def initbuf(i, carry):
        onesv[pl.ds(i * 16, 16)] = jnp.ones((16,), jnp.float32)
        zerosv[pl.ds(i * 16, 16)] = jnp.zeros((16,), jnp.float32)
        return carry
    lax.fori_loop(0, EPW // 16, initbuf, 0)

    _full, _rem = PER_TILE_W // EPW, PER_TILE_W % EPW
    for i in range(_full):
        pltpu.sync_copy(zerosv,
                        blk_sh.at[pl.ds(s * PER_TILE_W + i * EPW, EPW)])
    if _rem:
        pltpu.sync_copy(zerosv.at[pl.ds(0, _rem)],
                        blk_sh.at[pl.ds(s * PER_TILE_W + _full * EPW, _rem)])
    pltpu.sync_copy(zerosv.at[pl.ds(0, NP // NTILES)],
                    deg_sh.at[pl.ds(s * (NP // NTILES), NP // NTILES)])
    plsc.subcore_barrier()

    @pl.when(c == 0)
    def _deg():
        pltpu.sync_copy(onesv, deg_sh.at[rowv], add=True)
        plsc.subcore_barrier()
        pltpu.sync_copy(deg_sh.at[pl.ds(s * (NP // NTILES), NP // NTILES)],
                        deg_out.at[pl.ds(s * (NP // NTILES), NP // NTILES)])

    def blk_body(j, carry):
        b = 2 * j + c
        base = b * R

        def idx_body(i, icarry):
            r16 = rowv[pl.ds(i * 16, 16)]
            c16 = colv[pl.ds(i * 16, 16)]
            inb = (r16 >= base) & (r16 < base + R)
            idxv[pl.ds(i * 16, 16)] = jnp.where(
                inb, (r16 - base) * NP + c16, R * NP + c16)
            return icarry
        lax.fori_loop(0, EPW // 16, idx_body, 0)

        pltpu.sync_copy(onesv, blk_sh.at[idxv], add=True)
        plsc.subcore_barrier()
        rpt = R // NTILES
        pltpu.sync_copy(blk_sh.at[pl.ds(s * rpt * NP, rpt * NP)],
                        c_out.at[pl.ds(base * NP + s * rpt * NP, rpt * NP)])
        plsc.subcore_barrier()
        pltpu.sync_copy(zerosv, blk_sh.at[idxv])
        plsc.subcore_barrier()
        return carry

    lax.fori_loop(0, NBLK // 2, blk_body, 0)


@jax.jit
def _sc_build(row, col):
    mesh = plsc.VectorSubcoreMesh(core_axis_name="c", subcore_axis_name="s")
    f = pl.kernel(
        _sc_body,
        out_type=[
            jax.ShapeDtypeStruct((NP * NP,), jnp.float32),
            jax.ShapeDtypeStruct((NP,), jnp.float32),
        ],
        mesh=mesh,
        scratch_types=[
            pltpu.VMEM((EPW,), jnp.int32),
            pltpu.VMEM((EPW,), jnp.int32),
            pltpu.VMEM((EPW,), jnp.int32),
            pltpu.VMEM((EPW,), jnp.float32),
            pltpu.VMEM((EPW,), jnp.float32),
            pltpu.VMEM_SHARED((BLOCKW,), jnp.float32),
            pltpu.VMEM_SHARED((NP,), jnp.float32),
        ],
    )
    return f(row, col)


BR = 256
NRB = NP // BR


def _tc_body(deg_ref, x_ref, c_ref, out_ref, g0, g1, dinv_ref):
    k = pl.program_id(0)
    r = pl.program_id(1)

    @pl.when((k == 0) & (r == 0))
    def _init():
        deg = deg_ref[...]
        dinv = jnp.where(deg > 0.0,
                         lax.rsqrt(jnp.maximum(deg, 1e-12)),
                         0.0)
        dinv_ref[...] = dinv
        g0[...] = (dinv * x_ref[...]).astype(jnp.bfloat16)

    def _step(gread, gwrite):
        blk = c_ref[...]
        srow = jnp.dot(blk, gread[...], preferred_element_type=jnp.float32)
        dr = dinv_ref[pl.ds(r * BR, BR), :]
        xb = x_ref[pl.ds(r * BR, BR), :]
        hb = BETA * (dr * srow) + ALPHA * xb
        out_ref[...] = hb * (1.0 / THETA)
        gwrite[pl.ds(r * BR, BR), :] = (dr * hb).astype(jnp.bfloat16)

    @pl.when(k % 2 == 0)
    def _even():
        _step(g0, g1)

    @pl.when(k % 2 == 1)
    def _odd():
        _step(g1, g0)


@jax.jit
def _tc_diffuse(deg, xp, cvec):
    return pl.pallas_call(
        _tc_body,
        grid=(K, NRB),
        in_specs=[
            pl.BlockSpec((NP, 1), lambda k, r: (0, 0)),
            pl.BlockSpec((NP, D), lambda k, r: (0, 0)),
            pl.BlockSpec((BR, NP), lambda k, r: (r, 0)),
        ],
        out_specs=pl.BlockSpec((BR, D), lambda k, r: (r, 0)),
        out_shape=jax.ShapeDtypeStruct((NP, D), jnp.float32),
        scratch_shapes=[
            pltpu.VMEM((NP, D), jnp.bfloat16),
            pltpu.VMEM((NP, D), jnp.bfloat16),
            pltpu.VMEM((NP, 1), jnp.float32),
        ],
        compiler_params=pltpu.CompilerParams(
            dimension_semantics=("arbitrary", "arbitrary"),
            vmem_limit_bytes=100 * 1024 * 1024,
        ),
    )(deg, xp, cvec)


def kernel(x, edge_index):
    row = edge_index[0]
    col = edge_index[1]
    xp = jnp.pad(x, ((0, NP - N), (0, 0)))
    c_flat, deg = _sc_build(row, col)
    cvec = jax.lax.optimization_barrier(
        c_flat.astype(jnp.bfloat16)).reshape(NP, NP)
    out = _tc_diffuse(deg.reshape(NP, 1), xp, cvec)
    return out[:N]


# speedup vs baseline: 1.0239x; 1.0239x over previous
import jax
import jax.numpy as jnp
from jax import lax
from jax.experimental import pallas as pl
from jax.experimental.pallas import tpu as pltpu
from jax.experimental.pallas import tpu_sc as plsc

N = 10000
E = 160000
D = 256
K = 10
ALPHA = 0.1
BETA = 0.9
_s = sum(BETA**j for j in range(K))
THETA = BETA**K + ALPHA * _s

NP = 10240
R = 80
NBLK = NP // R
NTILES = 16
EPW = E // NTILES
BLOCKW = (R + 1) * NP
PER_TILE_W = BLOCKW // NTILES


def _sc_body(row_hbm, col_hbm, c_out, deg_out, rowv, colv, idxv, onesv,
             zerosv, blk_sh, deg_sh):
    c = lax.axis_index("c")
    s = lax.axis_index("s")

    pltpu.sync_copy(row_hbm.at[pl.ds(s * EPW, EPW)], rowv)
    pltpu.sync_copy(col_hbm.at[pl.ds(s * EPW, EPW)], colv)

    def initbuf(i, carry):
        onesv[pl.ds(i * 16, 16)] = jnp.ones((16,), jnp.float32)
        zerosv[pl.ds(i * 16, 16)] = jnp.zeros((16,), jnp.float32)
        return carry
    lax.fori_loop(0, EPW // 16, initbuf, 0)

    _full, _rem = PER_TILE_W // EPW, PER_TILE_W % EPW
    for i in range(_full):
        pltpu.sync_copy(zerosv,
                        blk_sh.at[pl.ds(s * PER_TILE_W + i * EPW, EPW)])
    if _rem:
        pltpu.sync_copy(zerosv.at[pl.ds(0, _rem)],
                        blk_sh.at[pl.ds(s * PER_TILE_W + _full * EPW, _rem)])
    pltpu.sync_copy(zerosv.at[pl.ds(0, NP // NTILES)],
                    deg_sh.at[pl.ds(s * (NP // NTILES), NP // NTILES)])
    plsc.subcore_barrier()

    @pl.when(c == 0)
    def _deg():
        pltpu.sync_copy(onesv, deg_sh.at[rowv], add=True)
        plsc.subcore_barrier()
        pltpu.sync_copy(deg_sh.at[pl.ds(s * (NP // NTILES), NP // NTILES)],
                        deg_out.at[pl.ds(s * (NP // NTILES), NP // NTILES)])

    def blk_body(j, carry):
        b = 2 * j + c
        base = b * R

        def idx_body(i, icarry):
            r16 = rowv[pl.ds(i * 16, 16)]
            c16 = colv[pl.ds(i * 16, 16)]
            inb = (r16 >= base) & (r16 < base + R)
            idxv[pl.ds(i * 16, 16)] = jnp.where(
                inb, (r16 - base) * NP + c16, R * NP + c16)
            return icarry
        lax.fori_loop(0, EPW // 16, idx_body, 0)

        pltpu.sync_copy(onesv, blk_sh.at[idxv], add=True)
        plsc.subcore_barrier()
        rpt = R // NTILES
        pltpu.sync_copy(blk_sh.at[pl.ds(s * rpt * NP, rpt * NP)],
                        c_out.at[pl.ds(base * NP + s * rpt * NP, rpt * NP)])
        plsc.subcore_barrier()
        pltpu.sync_copy(zerosv, blk_sh.at[idxv])
        plsc.subcore_barrier()
        return carry

    lax.fori_loop(0, NBLK // 2, blk_body, 0)


@jax.jit
def _sc_build(row, col):
    mesh = plsc.VectorSubcoreMesh(core_axis_name="c", subcore_axis_name="s")
    f = pl.kernel(
        _sc_body,
        out_type=[
            jax.ShapeDtypeStruct((NP * NP,), jnp.float32),
            jax.ShapeDtypeStruct((NP,), jnp.float32),
        ],
        mesh=mesh,
        scratch_types=[
            pltpu.VMEM((EPW,), jnp.int32),
            pltpu.VMEM((EPW,), jnp.int32),
            pltpu.VMEM((EPW,), jnp.int32),
            pltpu.VMEM((EPW,), jnp.float32),
            pltpu.VMEM((EPW,), jnp.float32),
            pltpu.VMEM_SHARED((BLOCKW,), jnp.float32),
            pltpu.VMEM_SHARED((NP,), jnp.float32),
        ],
    )
    return f(row, col)


BR = 256
NRB = NP // BR


def _tc_body(deg_ref, x_ref, c_ref, out_ref, g0, g1, dinv_ref):
    k = pl.program_id(0)
    r = pl.program_id(1)

    @pl.when((k == 0) & (r == 0))
    def _init():
        deg = deg_ref[...]
        dinv = jnp.where(deg > 0.0,
                         lax.rsqrt(jnp.maximum(deg, 1e-12)),
                         0.0)
        dinv_ref[...] = dinv
        g0[...] = (dinv * x_ref[...]).astype(jnp.bfloat16)

    def _step(gread, gwrite):
        blk = c_ref[...].astype(jnp.bfloat16)
        srow = jnp.dot(blk, gread[...], preferred_element_type=jnp.float32)
        dr = dinv_ref[pl.ds(r * BR, BR), :]
        xb = x_ref[pl.ds(r * BR, BR), :]
        hb = BETA * (dr * srow) + ALPHA * xb
        out_ref[...] = hb * (1.0 / THETA)
        gwrite[pl.ds(r * BR, BR), :] = (dr * hb).astype(jnp.bfloat16)

    @pl.when(k % 2 == 0)
    def _even():
        _step(g0, g1)

    @pl.when(k % 2 == 1)
    def _odd():
        _step(g1, g0)


@jax.jit
def _tc_diffuse(deg, xp, cmat):
    return pl.pallas_call(
        _tc_body,
        grid=(K, NRB),
        in_specs=[
            pl.BlockSpec((NP, 1), lambda k, r: (0, 0)),
            pl.BlockSpec((NP, D), lambda k, r: (0, 0)),
            pl.BlockSpec((BR, NP), lambda k, r: (r, 0)),
        ],
        out_specs=pl.BlockSpec((BR, D), lambda k, r: (r, 0)),
        out_shape=jax.ShapeDtypeStruct((NP, D), jnp.float32),
        scratch_shapes=[
            pltpu.VMEM((NP, D), jnp.bfloat16),
            pltpu.VMEM((NP, D), jnp.bfloat16),
            pltpu.VMEM((NP, 1), jnp.float32),
        ],
        compiler_params=pltpu.CompilerParams(
            dimension_semantics=("arbitrary", "arbitrary"),
            vmem_limit_bytes=100 * 1024 * 1024,
        ),
    )(deg, xp, cmat)


def kernel(x, edge_index):
    row = edge_index[0]
    col = edge_index[1]
    xp = jnp.pad(x, ((0, NP - N), (0, 0)))
    c_flat, deg = _sc_build(row, col)
    cvec = jax.lax.optimization_barrier(
        c_flat.astype(jnp.float8_e4m3fn)).reshape(NP, NP)
    out = _tc_diffuse(deg.reshape(NP, 1), xp, cvec)
    return out[:N]


# speedup vs baseline: 1.2702x; 1.2406x over previous
import jax
import jax.numpy as jnp
from jax import lax
from jax.experimental import pallas as pl
from jax.experimental.pallas import tpu as pltpu
from jax.experimental.pallas import tpu_sc as plsc

N = 10000
E = 160000
D = 256
K = 10
ALPHA = 0.1
BETA = 0.9
_s = sum(BETA**j for j in range(K))
THETA = BETA**K + ALPHA * _s

NP = 10240
R = 80
NBLK = NP // R
NTILES = 16
EPW = E // NTILES
BLOCKW = (R + 1) * NP
PER_TILE_W = BLOCKW // NTILES


def _sc_body(row_hbm, col_hbm, c_out, deg_out, rowv, colv, idxv, onesv,
             zerosv, blk_sh, deg_sh):
    c = lax.axis_index("c")
    s = lax.axis_index("s")

    pltpu.sync_copy(row_hbm.at[pl.ds(s * EPW, EPW)], rowv)
    pltpu.sync_copy(col_hbm.at[pl.ds(s * EPW, EPW)], colv)

    def initbuf(i, carry):
        onesv[pl.ds(i * 16, 16)] = jnp.ones((16,), jnp.float32)
        zerosv[pl.ds(i * 16, 16)] = jnp.zeros((16,), jnp.float32)
        return carry
    lax.fori_loop(0, EPW // 16, initbuf, 0)

    _full, _rem = PER_TILE_W // EPW, PER_TILE_W % EPW
    for i in range(_full):
        pltpu.sync_copy(zerosv,
                        blk_sh.at[pl.ds(s * PER_TILE_W + i * EPW, EPW)])
    if _rem:
        pltpu.sync_copy(zerosv.at[pl.ds(0, _rem)],
                        blk_sh.at[pl.ds(s * PER_TILE_W + _full * EPW, _rem)])
    pltpu.sync_copy(zerosv.at[pl.ds(0, NP // NTILES)],
                    deg_sh.at[pl.ds(s * (NP // NTILES), NP // NTILES)])
    plsc.subcore_barrier()

    @pl.when(c == 0)
    def _deg():
        pltpu.sync_copy(onesv, deg_sh.at[rowv], add=True)
        plsc.subcore_barrier()
        pltpu.sync_copy(deg_sh.at[pl.ds(s * (NP // NTILES), NP // NTILES)],
                        deg_out.at[pl.ds(s * (NP // NTILES), NP // NTILES)])

    def blk_body(j, carry):
        b = 2 * j + c
        base = b * R

        def idx_body(i, icarry):
            r16 = rowv[pl.ds(i * 16, 16)]
            c16 = colv[pl.ds(i * 16, 16)]
            inb = (r16 >= base) & (r16 < base + R)
            idxv[pl.ds(i * 16, 16)] = jnp.where(
                inb, (r16 - base) * NP + c16, R * NP + c16)
            return icarry
        lax.fori_loop(0, EPW // 16, idx_body, 0)

        pltpu.sync_copy(onesv, blk_sh.at[idxv], add=True)
        plsc.subcore_barrier()
        rpt = R // NTILES
        for rr in range(rpt):
            pltpu.sync_copy(blk_sh.at[pl.ds((s * rpt + rr) * NP, NP)],
                            c_out.at[base + s * rpt + rr])
        plsc.subcore_barrier()
        pltpu.sync_copy(zerosv, blk_sh.at[idxv])
        plsc.subcore_barrier()
        return carry

    lax.fori_loop(0, NBLK // 2, blk_body, 0)


@jax.jit
def _sc_build(row, col):
    mesh = plsc.VectorSubcoreMesh(core_axis_name="c", subcore_axis_name="s")
    f = pl.kernel(
        _sc_body,
        out_type=[
            jax.ShapeDtypeStruct((NP, NP), jnp.float32),
            jax.ShapeDtypeStruct((NP,), jnp.float32),
        ],
        mesh=mesh,
        scratch_types=[
            pltpu.VMEM((EPW,), jnp.int32),
            pltpu.VMEM((EPW,), jnp.int32),
            pltpu.VMEM((EPW,), jnp.int32),
            pltpu.VMEM((EPW,), jnp.float32),
            pltpu.VMEM((EPW,), jnp.float32),
            pltpu.VMEM_SHARED((BLOCKW,), jnp.float32),
            pltpu.VMEM_SHARED((NP,), jnp.float32),
        ],
    )
    return f(row, col)


BR = 256
NRB = NP // BR


def _tc_body(deg_ref, x_ref, c_ref, out_ref, g0, g1, dinv_ref):
    k = pl.program_id(0)
    r = pl.program_id(1)

    @pl.when((k == 0) & (r == 0))
    def _init():
        deg = deg_ref[...]
        dinv = jnp.where(deg > 0.0,
                         lax.rsqrt(jnp.maximum(deg, 1e-12)),
                         0.0)
        dinv_ref[...] = dinv
        g0[...] = (dinv * x_ref[...]).astype(jnp.bfloat16)

    def _step(gread, gwrite):
        blk = c_ref[...].astype(jnp.bfloat16)
        srow = jnp.dot(blk, gread[...], preferred_element_type=jnp.float32)
        dr = dinv_ref[pl.ds(r * BR, BR), :]
        xb = x_ref[pl.ds(r * BR, BR), :]
        hb = BETA * (dr * srow) + ALPHA * xb
        out_ref[...] = hb * (1.0 / THETA)
        gwrite[pl.ds(r * BR, BR), :] = (dr * hb).astype(jnp.bfloat16)

    @pl.when(k % 2 == 0)
    def _even():
        _step(g0, g1)

    @pl.when(k % 2 == 1)
    def _odd():
        _step(g1, g0)


@jax.jit
def _tc_diffuse(deg, xp, cmat):
    return pl.pallas_call(
        _tc_body,
        grid=(K, NRB),
        in_specs=[
            pl.BlockSpec((NP, 1), lambda k, r: (0, 0)),
            pl.BlockSpec((NP, D), lambda k, r: (0, 0)),
            pl.BlockSpec((BR, NP), lambda k, r: (r, 0)),
        ],
        out_specs=pl.BlockSpec((BR, D), lambda k, r: (r, 0)),
        out_shape=jax.ShapeDtypeStruct((NP, D), jnp.float32),
        scratch_shapes=[
            pltpu.VMEM((NP, D), jnp.bfloat16),
            pltpu.VMEM((NP, D), jnp.bfloat16),
            pltpu.VMEM((NP, 1), jnp.float32),
        ],
        compiler_params=pltpu.CompilerParams(
            dimension_semantics=("arbitrary", "arbitrary"),
            vmem_limit_bytes=100 * 1024 * 1024,
        ),
    )(deg, xp, cmat)


def kernel(x, edge_index):
    row = edge_index[0]
    col = edge_index[1]
    xp = jnp.pad(x, ((0, NP - N), (0, 0)))
    c2d, deg = _sc_build(row, col)
    cvec = c2d.astype(jnp.float8_e4m3fn)
    out = _tc_diffuse(deg.reshape(NP, 1), xp, cvec)
    return out[:N]


# speedup vs baseline: 1.2705x; 1.0003x over previous
import jax
import jax.numpy as jnp
from jax import lax
from jax.experimental import pallas as pl
from jax.experimental.pallas import tpu as pltpu
from jax.experimental.pallas import tpu_sc as plsc

N = 10000
E = 160000
D = 256
K = 10
ALPHA = 0.1
BETA = 0.9
_s = sum(BETA**j for j in range(K))
THETA = BETA**K + ALPHA * _s

NP = 10240
R = 80
NBLK = NP // R
NTILES = 16
EPW = E // NTILES
BLOCKW = (R + 1) * NP
PER_TILE_W = BLOCKW // NTILES


def _sc_body(row_hbm, col_hbm, c_out, deg_out, rowv, colv, idxv, onesv,
             zerosv, blk_sh, deg_sh):
    c = lax.axis_index("c")
    s = lax.axis_index("s")

    pltpu.sync_copy(row_hbm.at[pl.ds(s * EPW, EPW)], rowv)
    pltpu.sync_copy(col_hbm.at[pl.ds(s * EPW, EPW)], colv)

    def initbuf(i, carry):
        onesv[pl.ds(i * 16, 16)] = jnp.ones((16,), jnp.float32)
        zerosv[pl.ds(i * 16, 16)] = jnp.zeros((16,), jnp.float32)
        return carry
    lax.fori_loop(0, EPW // 16, initbuf, 0)

    _full, _rem = PER_TILE_W // EPW, PER_TILE_W % EPW
    for i in range(_full):
        pltpu.sync_copy(zerosv,
                        blk_sh.at[pl.ds(s * PER_TILE_W + i * EPW, EPW)])
    if _rem:
        pltpu.sync_copy(zerosv.at[pl.ds(0, _rem)],
                        blk_sh.at[pl.ds(s * PER_TILE_W + _full * EPW, _rem)])
    pltpu.sync_copy(zerosv.at[pl.ds(0, NP // NTILES)],
                    deg_sh.at[pl.ds(s * (NP // NTILES), NP // NTILES)])
    plsc.subcore_barrier()

    @pl.when(c == 0)
    def _deg():
        pltpu.sync_copy(onesv, deg_sh.at[rowv], add=True)
        plsc.subcore_barrier()
        pltpu.sync_copy(deg_sh.at[pl.ds(s * (NP // NTILES), NP // NTILES)],
                        deg_out.at[pl.ds(s * (NP // NTILES), NP // NTILES)])

    def blk_body(j, carry):
        b = 2 * j + c
        base = b * R

        def idx_body(i, icarry):
            r16 = rowv[pl.ds(i * 16, 16)]
            c16 = colv[pl.ds(i * 16, 16)]
            inb = (r16 >= base) & (r16 < base + R)
            idxv[pl.ds(i * 16, 16)] = jnp.where(
                inb, (r16 - base) * NP + c16, R * NP + c16)
            return icarry
        lax.fori_loop(0, EPW // 16, idx_body, 0)

        pltpu.sync_copy(onesv, blk_sh.at[idxv], add=True)
        plsc.subcore_barrier()
        rpt = R // NTILES
        for rr in range(rpt):
            pltpu.sync_copy(blk_sh.at[pl.ds((s * rpt + rr) * NP, NP)],
                            c_out.at[base + s * rpt + rr])
        plsc.subcore_barrier()
        pltpu.sync_copy(zerosv, blk_sh.at[idxv])
        plsc.subcore_barrier()
        return carry

    lax.fori_loop(0, NBLK // 2, blk_body, 0)


@jax.jit
def _sc_build(row, col):
    mesh = plsc.VectorSubcoreMesh(core_axis_name="c", subcore_axis_name="s")
    f = pl.kernel(
        _sc_body,
        out_type=[
            jax.ShapeDtypeStruct((NP, NP), jnp.float32),
            jax.ShapeDtypeStruct((NP,), jnp.float32),
        ],
        mesh=mesh,
        scratch_types=[
            pltpu.VMEM((EPW,), jnp.int32),
            pltpu.VMEM((EPW,), jnp.int32),
            pltpu.VMEM((EPW,), jnp.int32),
            pltpu.VMEM((EPW,), jnp.float32),
            pltpu.VMEM((EPW,), jnp.float32),
            pltpu.VMEM_SHARED((BLOCKW,), jnp.float32),
            pltpu.VMEM_SHARED((NP,), jnp.float32),
        ],
    )
    return f(row, col)


BR = 256
NRB = NP // BR


def _tc_body(deg_ref, x_ref, c_ref, out_ref, g0, g1, dinv_ref):
    k = pl.program_id(0)
    r = pl.program_id(1)

    @pl.when((k == 0) & (r == 0))
    def _init():
        deg = deg_ref[...]
        dinv = jnp.where(deg > 0.0,
                         lax.rsqrt(jnp.maximum(deg, 1e-12)),
                         0.0)
        dinv_ref[...] = dinv
        g0[...] = (dinv * x_ref[...]).astype(jnp.bfloat16)

    def _step(gread, gwrite):
        blk = c_ref[...]
        srow = jnp.dot(blk, gread[...], preferred_element_type=jnp.float32)
        dr = dinv_ref[pl.ds(r * BR, BR), :]
        xb = x_ref[pl.ds(r * BR, BR), :]
        hb = BETA * (dr * srow) + ALPHA * xb
        out_ref[...] = hb * (1.0 / THETA)
        gwrite[pl.ds(r * BR, BR), :] = (dr * hb).astype(jnp.bfloat16)

    @pl.when(k % 2 == 0)
    def _even():
        _step(g0, g1)

    @pl.when(k % 2 == 1)
    def _odd():
        _step(g1, g0)


@jax.jit
def _tc_diffuse(deg, xp, cmat):
    return pl.pallas_call(
        _tc_body,
        grid=(K, NRB),
        in_specs=[
            pl.BlockSpec((NP, 1), lambda k, r: (0, 0)),
            pl.BlockSpec((NP, D), lambda k, r: (0, 0)),
            pl.BlockSpec((BR, NP), lambda k, r: (r, 0)),
        ],
        out_specs=pl.BlockSpec((BR, D), lambda k, r: (r, 0)),
        out_shape=jax.ShapeDtypeStruct((NP, D), jnp.float32),
        scratch_shapes=[
            pltpu.VMEM((NP, D), jnp.bfloat16),
            pltpu.VMEM((NP, D), jnp.bfloat16),
            pltpu.VMEM((NP, 1), jnp.float32),
        ],
        compiler_params=pltpu.CompilerParams(
            dimension_semantics=("arbitrary", "arbitrary"),
            vmem_limit_bytes=100 * 1024 * 1024,
        ),
    )(deg, xp, cmat)


def kernel(x, edge_index):
    row = edge_index[0]
    col = edge_index[1]
    xp = jnp.pad(x, ((0, NP - N), (0, 0)))
    c2d, deg = _sc_build(row, col)
    cvec = c2d.astype(jnp.float8_e4m3fn)
    out = _tc_diffuse(deg.reshape(NP, 1), xp, cvec)
    return out[:N]
